# Initial kernel scaffold; baseline (speedup 1.0000x reference)
import jax
import jax.numpy as jnp
from jax.experimental import pallas as pl


def kernel(x, H, W1, b1, W2, b2, Wf, bf):
    raise NotImplementedError("write your pallas kernel here")


# speedup vs baseline: 3.7241x; 3.7241x over previous
import functools

import jax
import jax.numpy as jnp
from jax import lax
from jax.experimental import pallas as pl
from jax.experimental.pallas import tpu as pltpu
from jax.experimental.pallas import tpu_sc as plsc

N_NODES = 10000
N_HYEDGES = 10000
E = 320000
D = 128
N_TARGET = 16

NC = 2
NS = 16
NW = NC * NS
EPW = E // NW
CH = 80
NCHUNK = EPW // CH

NPAD = 10240
ROWS_PER_TILE = NPAD // NS

CW = 128

_mesh = plsc.VectorSubcoreMesh(core_axis_name="c", subcore_axis_name="s")


@functools.partial(
    pl.kernel,
    out_type=jax.ShapeDtypeStruct((NC * NPAD, D), jnp.float32),
    mesh=_mesh,
    scratch_types=[
        pltpu.VMEM((CH,), jnp.int32),
        pltpu.VMEM((CH,), jnp.int32),
        pltpu.VMEM((CH, D), jnp.float32),
        pltpu.VMEM_SHARED((NPAD, D), jnp.float32),
        pltpu.SemaphoreType.DMA,
    ],
)
def _sc_seg_sum(table, src, dst, zeros, out, idx_s, idx_d, rows, acc, sem):
    c = lax.axis_index("c")
    s = lax.axis_index("s")
    wid = c * NS + s
    base = wid * EPW

    row0 = s * ROWS_PER_TILE
    pltpu.sync_copy(zeros, acc.at[pl.ds(row0, ROWS_PER_TILE)])
    plsc.subcore_barrier()

    def body(g, carry):
        off = base + g * CH
        pltpu.sync_copy(src.at[pl.ds(off, CH)], idx_s)
        pltpu.sync_copy(dst.at[pl.ds(off, CH)], idx_d)
        pltpu.async_copy(table.at[idx_s], rows, sem).wait()
        pltpu.sync_copy(rows, acc.at[idx_d], add=True)
        return carry

    lax.fori_loop(0, NCHUNK, body, 0)

    plsc.subcore_barrier()
    pltpu.sync_copy(acc.at[pl.ds(row0, ROWS_PER_TILE)],
                    out.at[pl.ds(c * NPAD + row0, ROWS_PER_TILE)])


EPT_CNT = E // NS
NCHUNK_CNT = EPT_CNT // CH


@functools.partial(
    pl.kernel,
    out_type=jax.ShapeDtypeStruct((NC * NPAD, CW), jnp.float32),
    mesh=_mesh,
    scratch_types=[
        pltpu.VMEM((CH,), jnp.int32),
        pltpu.VMEM((CH, CW), jnp.float32),
        pltpu.VMEM_SHARED((NPAD, CW), jnp.float32),
    ],
)
def _sc_counts(hflat, ones, zeros, out, idx_d, ones_v, cnt):
    c = lax.axis_index("c")
    s = lax.axis_index("s")

    row0 = s * ROWS_PER_TILE
    pltpu.sync_copy(zeros, cnt.at[pl.ds(row0, ROWS_PER_TILE)])
    pltpu.sync_copy(ones, ones_v)
    plsc.subcore_barrier()

    base = c * E + s * EPT_CNT

    def body(g, carry):
        off = base + g * CH
        pltpu.sync_copy(hflat.at[pl.ds(off, CH)], idx_d)
        pltpu.sync_copy(ones_v, cnt.at[idx_d], add=True)
        return carry

    lax.fori_loop(0, NCHUNK_CNT, body, 0)

    plsc.subcore_barrier()
    pltpu.sync_copy(cnt.at[pl.ds(row0, ROWS_PER_TILE)],
                    out.at[pl.ds(c * NPAD + row0, ROWS_PER_TILE)])


BR = 1000
GRID = N_NODES // BR


def _mm_body(x_ref, w_ref, o_ref):
    o_ref[...] = jnp.dot(x_ref[...], w_ref[...],
                         preferred_element_type=jnp.float32)


def _tc_matmul(x, w):
    return pl.pallas_call(
        _mm_body,
        grid=(GRID,),
        in_specs=[pl.BlockSpec((BR, D), lambda i: (i, 0)),
                  pl.BlockSpec((D, D), lambda i: (0, 0))],
        out_specs=pl.BlockSpec((BR, D), lambda i: (i, 0)),
        out_shape=jax.ShapeDtypeStruct((N_NODES, D), jnp.float32),
    )(x, w)


def _comb_body(p0_ref, p1_ref, cnt_ref, o_ref):
    den = jnp.maximum(cnt_ref[...][:, :1], 1.0)
    o_ref[...] = (p0_ref[...] + p1_ref[...]) / den


def _tc_combine(p0, p1, cnt):
    return pl.pallas_call(
        _comb_body,
        grid=(GRID,),
        in_specs=[pl.BlockSpec((BR, D), lambda i: (i, 0)),
                  pl.BlockSpec((BR, D), lambda i: (i, 0)),
                  pl.BlockSpec((BR, CW), lambda i: (i, 0))],
        out_specs=pl.BlockSpec((BR, D), lambda i: (i, 0)),
        out_shape=jax.ShapeDtypeStruct((N_NODES, D), jnp.float32),
    )(p0, p1, cnt)


def _leaky(x):
    return jnp.where(x >= 0, x, 0.01 * x)


def _comb_mm_body(p0_ref, p1_ref, cnt_ref, b_ref, w_ref, o_ref):
    den = jnp.maximum(cnt_ref[...][:, :1], 1.0)
    h = _leaky((p0_ref[...] + p1_ref[...]) / den + b_ref[...])
    o_ref[...] = jnp.dot(h, w_ref[...], preferred_element_type=jnp.float32)


def _tc_combine_matmul(p0, p1, cnt, b, w):
    return pl.pallas_call(
        _comb_mm_body,
        grid=(GRID,),
        in_specs=[pl.BlockSpec((BR, D), lambda i: (i, 0)),
                  pl.BlockSpec((BR, D), lambda i: (i, 0)),
                  pl.BlockSpec((BR, CW), lambda i: (i, 0)),
                  pl.BlockSpec((1, D), lambda i: (0, 0)),
                  pl.BlockSpec((D, D), lambda i: (0, 0))],
        out_specs=pl.BlockSpec((BR, D), lambda i: (i, 0)),
        out_shape=jax.ShapeDtypeStruct((N_NODES, D), jnp.float32),
    )(p0, p1, cnt, b, w)


def _final_body(p0_ref, p1_ref, cnt_ref, b_ref, wf_ref, bf_ref,
                feats_ref, fp_ref, out_ref):
    i = pl.program_id(0)
    den = jnp.maximum(cnt_ref[...][:, :1], 1.0)
    h = _leaky((p0_ref[...] + p1_ref[...]) / den + b_ref[...])
    feats_ref[...] = h
    part = jnp.sum(h, axis=0, keepdims=True)

    @pl.when(i == 0)
    def _():
        fp_ref[...] = jnp.zeros_like(fp_ref)
        out_ref[...] = jnp.zeros_like(out_ref)

    fp_ref[...] += part

    @pl.when(i == GRID - 1)
    def _():
        fp = fp_ref[...] / float(N_NODES)
        fp_ref[...] = fp
        logits = jnp.dot(fp, wf_ref[...],
                         preferred_element_type=jnp.float32) + bf_ref[...]
        out_ref[...] = 1.0 / (1.0 + jnp.exp(-logits))


def _tc_final(p0, p1, cnt, b, wf, bf):
    return pl.pallas_call(
        _final_body,
        grid=(GRID,),
        in_specs=[pl.BlockSpec((BR, D), lambda i: (i, 0)),
                  pl.BlockSpec((BR, D), lambda i: (i, 0)),
                  pl.BlockSpec((BR, CW), lambda i: (i, 0)),
                  pl.BlockSpec((1, D), lambda i: (0, 0)),
                  pl.BlockSpec((D, N_TARGET), lambda i: (0, 0)),
                  pl.BlockSpec((1, N_TARGET), lambda i: (0, 0))],
        out_specs=[pl.BlockSpec((BR, D), lambda i: (i, 0)),
                   pl.BlockSpec((1, D), lambda i: (0, 0)),
                   pl.BlockSpec((1, N_TARGET), lambda i: (0, 0))],
        out_shape=[jax.ShapeDtypeStruct((N_NODES, D), jnp.float32),
                   jax.ShapeDtypeStruct((1, D), jnp.float32),
                   jax.ShapeDtypeStruct((1, N_TARGET), jnp.float32)],
    )(p0, p1, cnt, b, wf, bf)


def kernel(x, H, W1, b1, W2, b2, Wf, bf):
    node_idx = H[0]
    hyedge_idx = H[1]
    zeros_d = jnp.zeros((ROWS_PER_TILE, D), jnp.float32)
    ones_c = jnp.ones((CH, CW), jnp.float32)

    counts = _sc_counts(H.reshape(2 * E), ones_c, zeros_d)
    dv = counts[:N_NODES]
    de = counts[NPAD:NPAD + N_NODES]

    y1 = _tc_matmul(x, W1)
    ep = _sc_seg_sum(y1, node_idx, hyedge_idx, zeros_d)
    e1 = _tc_combine(ep[:N_NODES], ep[NPAD:NPAD + N_NODES], de)
    np_ = _sc_seg_sum(e1, hyedge_idx, node_idx, zeros_d)
    y2 = _tc_combine_matmul(np_[:N_NODES], np_[NPAD:NPAD + N_NODES], dv,
                            b1.reshape(1, D), W2)
    ep2 = _sc_seg_sum(y2, node_idx, hyedge_idx, zeros_d)
    e2 = _tc_combine(ep2[:N_NODES], ep2[NPAD:NPAD + N_NODES], de)
    np2 = _sc_seg_sum(e2, hyedge_idx, node_idx, zeros_d)
    feats, fp, out = _tc_final(np2[:N_NODES], np2[NPAD:NPAD + N_NODES], dv,
                               b2.reshape(1, D), Wf, bf.reshape(1, N_TARGET))
    return (out.reshape(N_TARGET), feats, fp)
